# baseline (speedup 1.0000x reference)
import functools
import jax
import jax.numpy as jnp
from jax import lax
from jax.experimental import pallas as pl
from jax.experimental.pallas import tpu as pltpu

_SCALE = 1.0 + 0.1
_N = 128
_F = 256
_CROWS = 32
_CHUNKS = _N // _CROWS


def _tc_body(sz_ref, x_ref, o_ref):
    t = pl.program_id(0)
    j = pl.program_id(1)
    s = sz_ref[t]
    rows = j * _CROWS + lax.broadcasted_iota(jnp.int32, (1, _CROWS, _F), 1)
    scale = jnp.where(rows < s, _SCALE, 0.0).astype(o_ref.dtype)
    o_ref[...] = x_ref[...] * scale


def _x_index_map(t, j, sz_ref):
    s = sz_ref[t]
    last_live = jnp.maximum((s + _CROWS - 1) // _CROWS - 1, 0)
    return (t, jnp.minimum(j, last_live), 0)


def kernel(input_tensor, batch_set_size):
    B, T, N, F = input_tensor.shape
    BT = B * T
    x = input_tensor.reshape(BT, N, F)
    sizes = batch_set_size.reshape(BT)

    grid_spec = pltpu.PrefetchScalarGridSpec(
        num_scalar_prefetch=1,
        grid=(BT, _CHUNKS),
        in_specs=[pl.BlockSpec((1, _CROWS, _F), _x_index_map)],
        out_specs=pl.BlockSpec((1, _CROWS, _F), lambda t, j, sz: (t, j, 0)),
    )
    out = pl.pallas_call(
        _tc_body,
        grid_spec=grid_spec,
        out_shape=jax.ShapeDtypeStruct((BT, N, F), input_tensor.dtype),
    )(sizes, x)
    return out.reshape(B, T, N, F)


# speedup vs baseline: 14.5230x; 14.5230x over previous
import jax
import jax.numpy as jnp
from jax import lax
from jax.experimental import pallas as pl
from jax.experimental.pallas import tpu as pltpu

_SCALE = 1.0 + 0.1
_N = 128
_F = 256
_CROWS = 32
_CPT = _N // _CROWS
_G = 8
_NG = 32
_GROWS = _G * _N


def _body(sz_ref, x_ref, o_ref, buf_a, buf_b, buf_c,
          sem_ra, sem_rb, sem_rc, sem_wa, sem_wb, sem_wc):
    bufs = (buf_a, buf_b, buf_c)
    rsems = (sem_ra, sem_rb, sem_rc)
    wsems = (sem_wa, sem_wb, sem_wc)

    def chunk_copies(g):
        buf, sem = bufs[g % 3], rsems[g % 3]
        for k in range(_G):
            t = g * _G + k
            s = sz_ref[t]
            for c in range(_CPT):
                yield s > c * _CROWS, pltpu.make_async_copy(
                    x_ref.at[pl.ds(t * _N + c * _CROWS, _CROWS)],
                    buf.at[pl.ds(k * _N + c * _CROWS, _CROWS)], sem)

    def read_group(g):
        for cond, copy in chunk_copies(g):
            @pl.when(cond)
            def _():
                copy.start()

    def wait_read_group(g):
        for cond, copy in chunk_copies(g):
            @pl.when(cond)
            def _():
                copy.wait()

    def compute(g):
        buf = bufs[g % 3]
        irows = lax.broadcasted_iota(jnp.int32, (_N, _F), 0)
        for k in range(_G):
            s = sz_ref[g * _G + k]
            scale = jnp.where(irows < s, _SCALE, 0.0).astype(jnp.float32)
            sl = pl.ds(k * _N, _N)
            buf[sl, :] = buf[sl, :] * scale

    def write_group(g):
        pltpu.make_async_copy(
            bufs[g % 3], o_ref.at[pl.ds(g * _GROWS, _GROWS)],
            wsems[g % 3]).start()

    def wait_write_group(g):
        pltpu.make_async_copy(
            bufs[g % 3], o_ref.at[pl.ds(g * _GROWS, _GROWS)],
            wsems[g % 3]).wait()

    read_group(0)
    for g in range(_NG):
        if g >= 2:
            wait_write_group(g - 2)
        if g + 1 < _NG:
            read_group(g + 1)
        wait_read_group(g)
        compute(g)
        write_group(g)
    wait_write_group(_NG - 2)
    wait_write_group(_NG - 1)


def kernel(input_tensor, batch_set_size):
    B, T, N, F = input_tensor.shape
    BT = B * T
    x = input_tensor.reshape(BT * N, F)
    sizes = batch_set_size.reshape(BT)

    out = pl.pallas_call(
        _body,
        in_specs=[
            pl.BlockSpec(memory_space=pltpu.MemorySpace.SMEM),
            pl.BlockSpec(memory_space=pltpu.MemorySpace.HBM),
        ],
        out_specs=pl.BlockSpec(memory_space=pltpu.MemorySpace.HBM),
        out_shape=jax.ShapeDtypeStruct((BT * N, F), input_tensor.dtype),
        scratch_shapes=[
            pltpu.VMEM((_GROWS, _F), jnp.float32),
            pltpu.VMEM((_GROWS, _F), jnp.float32),
            pltpu.VMEM((_GROWS, _F), jnp.float32),
            pltpu.SemaphoreType.DMA,
            pltpu.SemaphoreType.DMA,
            pltpu.SemaphoreType.DMA,
            pltpu.SemaphoreType.DMA,
            pltpu.SemaphoreType.DMA,
            pltpu.SemaphoreType.DMA,
        ],
    )(sizes, x)
    return out.reshape(B, T, N, F)


# speedup vs baseline: 18.8092x; 1.2951x over previous
import jax
import jax.numpy as jnp
from jax import lax
from jax.experimental import pallas as pl
from jax.experimental.pallas import tpu as pltpu

_SCALE = 1.0 + 0.1
_N = 128
_F = 256
_CROWS = 32
_CPT = _N // _CROWS
_G = 16
_NG = 16
_GROWS = _G * _N


def _body(sz_ref, x_ref, o_ref, buf_a, buf_b, buf_c,
          sem_ra, sem_rb, sem_rc, sem_wa, sem_wb, sem_wc):
    bufs = (buf_a, buf_b, buf_c)
    rsems = (sem_ra, sem_rb, sem_rc)
    wsems = (sem_wa, sem_wb, sem_wc)

    def chunk_copies(g):
        buf, sem = bufs[g % 3], rsems[g % 3]
        for k in range(_G):
            t = g * _G + k
            s = sz_ref[t]
            for c in range(_CPT):
                yield s > c * _CROWS, pltpu.make_async_copy(
                    x_ref.at[pl.ds(t * _N + c * _CROWS, _CROWS)],
                    buf.at[pl.ds(k * _N + c * _CROWS, _CROWS)], sem)

    def read_group(g):
        for cond, copy in chunk_copies(g):
            @pl.when(cond)
            def _():
                copy.start()

    def wait_read_group(g):
        for cond, copy in chunk_copies(g):
            @pl.when(cond)
            def _():
                copy.wait()

    def compute(g):
        buf = bufs[g % 3]
        irows = lax.broadcasted_iota(jnp.int32, (_N, _F), 0)
        for k in range(_G):
            s = sz_ref[g * _G + k]
            scale = jnp.where(irows < s, _SCALE, 0.0).astype(jnp.float32)
            sl = pl.ds(k * _N, _N)
            buf[sl, :] = buf[sl, :] * scale

    def write_group(g):
        pltpu.make_async_copy(
            bufs[g % 3], o_ref.at[pl.ds(g * _GROWS, _GROWS)],
            wsems[g % 3]).start()

    def wait_write_group(g):
        pltpu.make_async_copy(
            bufs[g % 3], o_ref.at[pl.ds(g * _GROWS, _GROWS)],
            wsems[g % 3]).wait()

    read_group(0)
    for g in range(_NG):
        if g >= 2:
            wait_write_group(g - 2)
        if g + 1 < _NG:
            read_group(g + 1)
        wait_read_group(g)
        compute(g)
        write_group(g)
    wait_write_group(_NG - 2)
    wait_write_group(_NG - 1)


def kernel(input_tensor, batch_set_size):
    B, T, N, F = input_tensor.shape
    BT = B * T
    x = input_tensor.reshape(BT * N, F)
    sizes = batch_set_size.reshape(BT)

    out = pl.pallas_call(
        _body,
        in_specs=[
            pl.BlockSpec(memory_space=pltpu.MemorySpace.SMEM),
            pl.BlockSpec(memory_space=pltpu.MemorySpace.HBM),
        ],
        out_specs=pl.BlockSpec(memory_space=pltpu.MemorySpace.HBM),
        out_shape=jax.ShapeDtypeStruct((BT * N, F), input_tensor.dtype),
        scratch_shapes=[
            pltpu.VMEM((_GROWS, _F), jnp.float32),
            pltpu.VMEM((_GROWS, _F), jnp.float32),
            pltpu.VMEM((_GROWS, _F), jnp.float32),
            pltpu.SemaphoreType.DMA,
            pltpu.SemaphoreType.DMA,
            pltpu.SemaphoreType.DMA,
            pltpu.SemaphoreType.DMA,
            pltpu.SemaphoreType.DMA,
            pltpu.SemaphoreType.DMA,
        ],
    )(sizes, x)
    return out.reshape(B, T, N, F)


# speedup vs baseline: 21.1936x; 1.1268x over previous
import jax
import jax.numpy as jnp
from jax import lax
from jax.experimental import pallas as pl
from jax.experimental.pallas import tpu as pltpu

_SCALE = 1.0 + 0.1
_N = 128
_F = 256
_CROWS = 32
_CPT = _N // _CROWS
_G = 32
_NG = 8
_GROWS = _G * _N


def _body(sz_ref, x_ref, o_ref, buf_a, buf_b, buf_c,
          sem_ra, sem_rb, sem_rc, sem_wa, sem_wb, sem_wc):
    bufs = (buf_a, buf_b, buf_c)
    rsems = (sem_ra, sem_rb, sem_rc)
    wsems = (sem_wa, sem_wb, sem_wc)

    def chunk_copies(g):
        buf, sem = bufs[g % 3], rsems[g % 3]
        for k in range(_G):
            t = g * _G + k
            s = sz_ref[t]
            for c in range(_CPT):
                yield s > c * _CROWS, pltpu.make_async_copy(
                    x_ref.at[pl.ds(t * _N + c * _CROWS, _CROWS)],
                    buf.at[pl.ds(k * _N + c * _CROWS, _CROWS)], sem)

    def read_group(g):
        for cond, copy in chunk_copies(g):
            @pl.when(cond)
            def _():
                copy.start()

    def wait_read_group(g):
        for cond, copy in chunk_copies(g):
            @pl.when(cond)
            def _():
                copy.wait()

    def compute(g):
        buf = bufs[g % 3]
        irows = lax.broadcasted_iota(jnp.int32, (_N, _F), 0)
        for k in range(_G):
            s = sz_ref[g * _G + k]
            scale = jnp.where(irows < s, _SCALE, 0.0).astype(jnp.float32)
            sl = pl.ds(k * _N, _N)
            buf[sl, :] = buf[sl, :] * scale

    def write_group(g):
        pltpu.make_async_copy(
            bufs[g % 3], o_ref.at[pl.ds(g * _GROWS, _GROWS)],
            wsems[g % 3]).start()

    def wait_write_group(g):
        pltpu.make_async_copy(
            bufs[g % 3], o_ref.at[pl.ds(g * _GROWS, _GROWS)],
            wsems[g % 3]).wait()

    read_group(0)
    for g in range(_NG):
        if g >= 2:
            wait_write_group(g - 2)
        if g + 1 < _NG:
            read_group(g + 1)
        wait_read_group(g)
        compute(g)
        write_group(g)
    wait_write_group(_NG - 2)
    wait_write_group(_NG - 1)


def kernel(input_tensor, batch_set_size):
    B, T, N, F = input_tensor.shape
    BT = B * T
    x = input_tensor.reshape(BT * N, F)
    sizes = batch_set_size.reshape(BT)

    out = pl.pallas_call(
        _body,
        in_specs=[
            pl.BlockSpec(memory_space=pltpu.MemorySpace.SMEM),
            pl.BlockSpec(memory_space=pltpu.MemorySpace.HBM),
        ],
        out_specs=pl.BlockSpec(memory_space=pltpu.MemorySpace.HBM),
        out_shape=jax.ShapeDtypeStruct((BT * N, F), input_tensor.dtype),
        scratch_shapes=[
            pltpu.VMEM((_GROWS, _F), jnp.float32),
            pltpu.VMEM((_GROWS, _F), jnp.float32),
            pltpu.VMEM((_GROWS, _F), jnp.float32),
            pltpu.SemaphoreType.DMA,
            pltpu.SemaphoreType.DMA,
            pltpu.SemaphoreType.DMA,
            pltpu.SemaphoreType.DMA,
            pltpu.SemaphoreType.DMA,
            pltpu.SemaphoreType.DMA,
        ],
    )(sizes, x)
    return out.reshape(B, T, N, F)


# speedup vs baseline: 22.6448x; 1.0685x over previous
import jax
import jax.numpy as jnp
from jax import lax
from jax.experimental import pallas as pl
from jax.experimental.pallas import tpu as pltpu

_SCALE = 1.0 + 0.1
_N = 128
_F = 256
_CROWS = 32
_CPT = _N // _CROWS
_G = 64
_NG = 4
_GROWS = _G * _N


def _body(sz_ref, x_ref, o_ref, buf_a, buf_b, buf_c,
          sem_ra, sem_rb, sem_rc, sem_wa, sem_wb, sem_wc):
    bufs = (buf_a, buf_b, buf_c)
    rsems = (sem_ra, sem_rb, sem_rc)
    wsems = (sem_wa, sem_wb, sem_wc)

    def chunk_copies(g):
        buf, sem = bufs[g % 3], rsems[g % 3]
        for k in range(_G):
            t = g * _G + k
            s = sz_ref[t]
            for c in range(_CPT):
                yield s > c * _CROWS, pltpu.make_async_copy(
                    x_ref.at[pl.ds(t * _N + c * _CROWS, _CROWS)],
                    buf.at[pl.ds(k * _N + c * _CROWS, _CROWS)], sem)

    def read_group(g):
        for cond, copy in chunk_copies(g):
            @pl.when(cond)
            def _():
                copy.start()

    def wait_read_group(g):
        for cond, copy in chunk_copies(g):
            @pl.when(cond)
            def _():
                copy.wait()

    def compute(g):
        buf = bufs[g % 3]
        irows = lax.broadcasted_iota(jnp.int32, (_N, _F), 0)
        for k in range(_G):
            s = sz_ref[g * _G + k]
            scale = jnp.where(irows < s, _SCALE, 0.0).astype(jnp.float32)
            sl = pl.ds(k * _N, _N)
            buf[sl, :] = buf[sl, :] * scale

    def write_group(g):
        pltpu.make_async_copy(
            bufs[g % 3], o_ref.at[pl.ds(g * _GROWS, _GROWS)],
            wsems[g % 3]).start()

    def wait_write_group(g):
        pltpu.make_async_copy(
            bufs[g % 3], o_ref.at[pl.ds(g * _GROWS, _GROWS)],
            wsems[g % 3]).wait()

    read_group(0)
    for g in range(_NG):
        if g >= 2:
            wait_write_group(g - 2)
        if g + 1 < _NG:
            read_group(g + 1)
        wait_read_group(g)
        compute(g)
        write_group(g)
    wait_write_group(_NG - 2)
    wait_write_group(_NG - 1)


def kernel(input_tensor, batch_set_size):
    B, T, N, F = input_tensor.shape
    BT = B * T
    x = input_tensor.reshape(BT * N, F)
    sizes = batch_set_size.reshape(BT)

    out = pl.pallas_call(
        _body,
        in_specs=[
            pl.BlockSpec(memory_space=pltpu.MemorySpace.SMEM),
            pl.BlockSpec(memory_space=pltpu.MemorySpace.HBM),
        ],
        out_specs=pl.BlockSpec(memory_space=pltpu.MemorySpace.HBM),
        out_shape=jax.ShapeDtypeStruct((BT * N, F), input_tensor.dtype),
        scratch_shapes=[
            pltpu.VMEM((_GROWS, _F), jnp.float32),
            pltpu.VMEM((_GROWS, _F), jnp.float32),
            pltpu.VMEM((_GROWS, _F), jnp.float32),
            pltpu.SemaphoreType.DMA,
            pltpu.SemaphoreType.DMA,
            pltpu.SemaphoreType.DMA,
            pltpu.SemaphoreType.DMA,
            pltpu.SemaphoreType.DMA,
            pltpu.SemaphoreType.DMA,
        ],
    )(sizes, x)
    return out.reshape(B, T, N, F)
